# baseline (speedup 1.0000x reference)
import functools

import jax
import jax.numpy as jnp
from jax import lax
from jax.experimental import pallas as pl
from jax.experimental.pallas import tpu as pltpu
from jax.experimental.pallas import tpu_sc as plsc

D = 128
B = 16384
K = 1000
NC = 2
NS = 16
NW = NC * NS
CHUNK = 128

NUM_CHUNKS = B // CHUNK
SC_CHUNKS = 64
CHUNKS_PER_W = SC_CHUNKS // NW

B_TC = (NUM_CHUNKS - SC_CHUNKS) * CHUNK
BM = 512
NB_TC = B_TC // BM


def _make_sc_kernel():
  mesh = plsc.VectorSubcoreMesh(core_axis_name="c", subcore_axis_name="s")

  @functools.partial(
      pl.kernel,
      mesh=mesh,
      out_type=jax.ShapeDtypeStruct((SC_CHUNKS, CHUNK, D), jnp.float32),
      scratch_types=[
          pltpu.VMEM((CHUNKS_PER_W, CHUNK), jnp.int32),
          pltpu.VMEM((CHUNKS_PER_W, CHUNK, D), jnp.float32),
          pltpu.SemaphoreType.DMA((CHUNKS_PER_W,)),
          pltpu.SemaphoreType.DMA,
      ],
  )
  def gather_kernel(table_hbm, idx_hbm, out_hbm, idx_v, rows_v, gsem, osem):
    wid = lax.axis_index("s") * NC + lax.axis_index("c")
    base = wid * CHUNKS_PER_W
    pltpu.sync_copy(idx_hbm.at[pl.ds(base, CHUNKS_PER_W)], idx_v)
    gathers = [
        pltpu.async_copy(table_hbm.at[idx_v.at[j]], rows_v.at[j], gsem.at[j])
        for j in range(CHUNKS_PER_W)
    ]
    writes = []
    for j in range(CHUNKS_PER_W):
      gathers[j].wait()
      writes.append(pltpu.async_copy(rows_v.at[j], out_hbm.at[base + j], osem))
    for c in writes:
      c.wait()

  return gather_kernel


_sc_gather = _make_sc_kernel()


def _tc_body(t_ref, table_ref, out_ref):
  tb = t_ref[0, 0, :]
  iota = lax.broadcasted_iota(jnp.int32, (BM, K), 1)
  oh = (tb[:, None] == iota).astype(jnp.float32)
  out_ref[...] = jnp.dot(oh, table_ref[...], preferred_element_type=jnp.float32)


def _tc_gather(idx_tc, table):
  return pl.pallas_call(
      _tc_body,
      grid=(NB_TC,),
      in_specs=[
          pl.BlockSpec((1, 1, BM), lambda i: (i, 0, 0)),
          pl.BlockSpec((K, D), lambda i: (0, 0)),
      ],
      out_specs=pl.BlockSpec((BM, D), lambda i: (i, 0)),
      out_shape=jax.ShapeDtypeStruct((B_TC, D), jnp.float32),
  )(idx_tc, table)


@jax.jit
def kernel(t, embedding_weight):
  idx = t.astype(jnp.int32)
  idx_sc = idx[: SC_CHUNKS * CHUNK].reshape(SC_CHUNKS, CHUNK)
  idx_tc = idx[SC_CHUNKS * CHUNK :].reshape(NB_TC, 1, BM)
  out_sc = _sc_gather(embedding_weight, idx_sc)
  out_tc = _tc_gather(idx_tc, embedding_weight)
  return jnp.concatenate(
      [out_sc.reshape(SC_CHUNKS * CHUNK, D), out_tc], axis=0
  )


# speedup vs baseline: 1.2435x; 1.2435x over previous
import functools

import jax
import jax.numpy as jnp
from jax import lax
from jax.experimental import pallas as pl
from jax.experimental.pallas import tpu as pltpu
from jax.experimental.pallas import tpu_sc as plsc

D = 128
B = 16384
NC = 2
NS = 16
CHUNK = 128
NUM_CHUNKS = B // CHUNK
N_CORE0 = 58
MAX_CPW = 5
WIN = 16
IDX_PAD = 136


def _make_kernel():
  mesh = plsc.VectorSubcoreMesh(core_axis_name="c", subcore_axis_name="s")

  @functools.partial(
      pl.kernel,
      mesh=mesh,
      out_type=jax.ShapeDtypeStruct((NUM_CHUNKS, CHUNK, D), jnp.float32),
      scratch_types=[
          pltpu.VMEM((WIN, CHUNK), jnp.int32),
          pltpu.VMEM((MAX_CPW, CHUNK, D), jnp.float32),
          pltpu.SemaphoreType.DMA((MAX_CPW,)),
          pltpu.SemaphoreType.DMA,
      ],
  )
  def gather_kernel(table_hbm, idx_hbm, out_hbm, idx_v, rows_v, gsem, osem):
    c = lax.axis_index("c")
    s = lax.axis_index("s")
    n_c = jnp.where(c == 0, N_CORE0, NUM_CHUNKS - N_CORE0)
    base_c = jnp.where(c == 0, 0, N_CORE0)
    lo = base_c + (s * n_c) // NS
    cnt = base_c + ((s + 1) * n_c) // NS - lo
    w8 = pl.multiple_of((lo // 8) * 8, 8)
    off = lo - w8
    pltpu.sync_copy(idx_hbm.at[pl.ds(w8, WIN)], idx_v)
    for j in range(MAX_CPW):
      @pl.when(j < cnt)
      def _issue(j=j):
        pltpu.async_copy(
            table_hbm.at[idx_v.at[off + j]], rows_v.at[j], gsem.at[j]
        )
    for j in range(MAX_CPW):
      @pl.when(j < cnt)
      def _drain(j=j):
        pltpu.make_async_copy(
            table_hbm.at[idx_v.at[off + j]], rows_v.at[j], gsem.at[j]
        ).wait()
        pltpu.async_copy(rows_v.at[j], out_hbm.at[lo + j], osem)
    for j in range(MAX_CPW):
      @pl.when(j < cnt)
      def _finish(j=j):
        pltpu.make_async_copy(rows_v.at[j], out_hbm.at[lo + j], osem).wait()

  return gather_kernel


_gather = _make_kernel()


@jax.jit
def kernel(t, embedding_weight):
  idx = t.astype(jnp.int32).reshape(NUM_CHUNKS, CHUNK)
  idx = jnp.pad(idx, ((0, IDX_PAD - NUM_CHUNKS), (0, 0)))
  out = _gather(embedding_weight, idx)
  return out.reshape(B, D)


# speedup vs baseline: 1.2981x; 1.0439x over previous
import functools

import jax
import jax.numpy as jnp
from jax import lax
from jax.experimental import pallas as pl
from jax.experimental.pallas import tpu as pltpu
from jax.experimental.pallas import tpu_sc as plsc

D = 128
B = 16384
NC = 2
NS = 16
NW = NC * NS
CHUNK = 128
ROWS_PER_W = B // NW
CHUNKS_PER_W = ROWS_PER_W // CHUNK
NUM_CHUNKS = B // CHUNK


def _make_kernel():
  mesh = plsc.VectorSubcoreMesh(core_axis_name="c", subcore_axis_name="s")

  @functools.partial(
      pl.kernel,
      mesh=mesh,
      out_type=jax.ShapeDtypeStruct((NUM_CHUNKS, CHUNK, D), jnp.float32),
      scratch_types=[
          pltpu.VMEM((CHUNKS_PER_W, CHUNK), jnp.int32),
          pltpu.VMEM((CHUNKS_PER_W, CHUNK, D), jnp.float32),
          pltpu.SemaphoreType.DMA,
      ],
  )
  def gather_kernel(table_hbm, idx_hbm, out_hbm, idx_v, rows_v, sem):
    wid = lax.axis_index("s") * NC + lax.axis_index("c")
    base = wid * CHUNKS_PER_W
    pltpu.sync_copy(idx_hbm.at[pl.ds(base, CHUNKS_PER_W)], idx_v)
    copies = [
        pltpu.async_copy(table_hbm.at[idx_v.at[j]], rows_v.at[j], sem)
        for j in range(CHUNKS_PER_W)
    ]
    for c in copies:
      c.wait()
    pltpu.sync_copy(rows_v, out_hbm.at[pl.ds(base, CHUNKS_PER_W)])

  return gather_kernel


_gather = _make_kernel()


@jax.jit
def kernel(t, embedding_weight):
  idx = t.astype(jnp.int32).reshape(NUM_CHUNKS, CHUNK)
  out = _gather(embedding_weight, idx)
  return out.reshape(B, D)


# speedup vs baseline: 1.3000x; 1.0015x over previous
import functools

import jax
import jax.numpy as jnp
from jax import lax
from jax.experimental import pallas as pl
from jax.experimental.pallas import tpu as pltpu
from jax.experimental.pallas import tpu_sc as plsc

D = 128
B = 16384
NC = 2
NS = 16
NW = NC * NS
CHUNK = 64
ROWS_PER_W = B // NW
CHUNKS_PER_W = ROWS_PER_W // CHUNK
NUM_CHUNKS = B // CHUNK


def _make_kernel():
  mesh = plsc.VectorSubcoreMesh(core_axis_name="c", subcore_axis_name="s")

  @functools.partial(
      pl.kernel,
      mesh=mesh,
      out_type=jax.ShapeDtypeStruct((NUM_CHUNKS, CHUNK, D), jnp.float32),
      scratch_types=[
          pltpu.VMEM((CHUNKS_PER_W, CHUNK), jnp.int32),
          pltpu.VMEM((CHUNKS_PER_W, CHUNK, D), jnp.float32),
          pltpu.SemaphoreType.DMA,
      ],
  )
  def gather_kernel(table_hbm, idx_hbm, out_hbm, idx_v, rows_v, sem):
    wid = lax.axis_index("s") * NC + lax.axis_index("c")
    base = wid * CHUNKS_PER_W
    pltpu.sync_copy(idx_hbm.at[pl.ds(base, CHUNKS_PER_W)], idx_v)
    copies = [
        pltpu.async_copy(table_hbm.at[idx_v.at[j]], rows_v.at[j], sem)
        for j in range(CHUNKS_PER_W)
    ]
    for c in copies:
      c.wait()
    pltpu.sync_copy(rows_v, out_hbm.at[pl.ds(base, CHUNKS_PER_W)])

  return gather_kernel


_gather = _make_kernel()


@jax.jit
def kernel(t, embedding_weight):
  idx = t.astype(jnp.int32).reshape(NUM_CHUNKS, CHUNK)
  out = _gather(embedding_weight, idx)
  return out.reshape(B, D)
